# Initial kernel scaffold; baseline (speedup 1.0000x reference)
import jax
import jax.numpy as jnp
from jax.experimental import pallas as pl


def kernel(x, edge_index, W1, b1, W2, b2, W3, b3, W4, b4, W5, b5):
    raise NotImplementedError("write your pallas kernel here")


# speedup vs baseline: 3050.4842x; 3050.4842x over previous
import jax
import jax.numpy as jnp
from jax import lax
from jax.experimental import pallas as pl

N = 1024


def _lrelu(t):
    return jnp.where(t >= 0, t, 0.01 * t)


def _fused(ei_ref, x_ref, W1_ref, b1_ref, W2_ref, b2_ref, W3_ref, b3_ref,
           W4_ref, b4_ref, W5_ref, b5_ref, recon_ref, xr_ref, z_ref):
    adj = (ei_ref[...] != 0).astype(jnp.float32)
    r = lax.broadcasted_iota(jnp.int32, (N, N), 0)
    c = lax.broadcasted_iota(jnp.int32, (N, N), 1)
    ahat = jnp.where(r == c, 1.0, adj)
    deg = jnp.sum(ahat, axis=0)
    dinv = jnp.where(deg > 0, lax.rsqrt(deg), 0.0)
    dcol = dinv[:, None]

    def conv(h, W_ref, b_ref):
        hw = jnp.dot(h, W_ref[...], preferred_element_type=jnp.float32)
        t = lax.dot_general(ahat, dcol * hw, (((0,), (0,)), ((), ())),
                            preferred_element_type=jnp.float32)
        return _lrelu(dcol * t + b_ref[...])

    h1 = conv(x_ref[...], W1_ref, b1_ref)
    z = conv(h1, W2_ref, b2_ref)
    re = conv(z, W3_ref, b3_ref)
    recon_ref[...] = jax.nn.sigmoid(
        lax.dot_general(re, re, (((1,), (1,)), ((), ())),
                        preferred_element_type=jnp.float32))
    xh = conv(z, W4_ref, b4_ref)
    xr_ref[...] = conv(xh, W5_ref, b5_ref)
    z_ref[...] = z


def kernel(x, edge_index, W1, b1, W2, b2, W3, b3, W4, b4, W5, b5):
    ei = edge_index.astype(jnp.int32)
    biases = [b.reshape(1, -1) for b in (b1, b2, b3, b4, b5)]
    out_shape = (
        jax.ShapeDtypeStruct((N, N), jnp.float32),
        jax.ShapeDtypeStruct((N, W5.shape[1]), jnp.float32),
        jax.ShapeDtypeStruct((N, W2.shape[1]), jnp.float32),
    )
    recon, xr, z = pl.pallas_call(
        _fused,
        out_shape=out_shape,
    )(ei, x, W1, biases[0], W2, biases[1], W3, biases[2],
      W4, biases[3], W5, biases[4])
    return (recon, xr, z)
